# baseline (speedup 1.0000x reference)
import functools

import jax
import jax.numpy as jnp
from jax import lax
from jax.experimental import pallas as pl
from jax.experimental.pallas import tpu as pltpu
from jax.experimental.pallas import tpu_sc as plsc

NUM_BATCHES = 8
NUM_FRAMES = 16
RES = 64
N = 2_000_000
NBF = NUM_BATCHES * NUM_FRAMES
TOTAL_CELLS = NBF * RES * RES * RES

BLK = 32768
BLK_ROWS = BLK // 128
ROWS = N // 128
GRID1 = 62
N_PAD = GRID1 * BLK

NW = 32
W_ELEMS = N_PAD // NW
CHUNK = 1024
W_CHUNKS = W_ELEMS // CHUNK
GATHERS = CHUNK // 128


def _phase1_body(kf_ref, p_ref, b_ref, t_ref, o_ref):
    def cellq(v):
        g = ((v / 2.0 + 0.5) * float(RES)).astype(jnp.int32)
        return jnp.clip(g, 0, RES - 1)

    gx = cellq(p_ref[0])
    gy = cellq(p_ref[1])
    gz = cellq(p_ref[2])
    spatial = (gx * (RES * RES) + gy * RES + gz).reshape(BLK)

    t = t_ref[...]
    cell = jnp.clip((t * float(NUM_FRAMES - 1)).astype(jnp.int32), 0,
                    NUM_FRAMES - 2)
    left = jnp.zeros_like(t)
    right = jnp.zeros_like(t)
    for i in range(NUM_FRAMES):
        ki = kf_ref[i]
        if i <= NUM_FRAMES - 2:
            left = jnp.where(cell == i, ki, left)
        if i >= 1:
            right = jnp.where(cell == i - 1, ki, right)
    fidx = cell + jnp.where(jnp.abs(t - left) <= jnp.abs(right - t), 0, 1)

    lin = spatial * NBF + b_ref[...] * NUM_FRAMES + fidx
    o_ref[...] = jnp.clip(lin, 0, TOTAL_CELLS - 1)


def _phase1(kf, pts3, bs, t):
    blk1 = lambda: pl.BlockSpec((BLK,), lambda i: (i,))
    return pl.pallas_call(
        _phase1_body,
        grid=(GRID1,),
        in_specs=[pl.BlockSpec(memory_space=pltpu.SMEM),
                  pl.BlockSpec((3, BLK_ROWS, 128), lambda i: (0, i, 0)),
                  blk1(), blk1()],
        out_specs=blk1(),
        out_shape=jax.ShapeDtypeStruct((N_PAD,), jnp.int32),
    )(kf, pts3, bs, t)


def _phase2_body(lin_hbm, occ_hbm, out_hbm,
                 idx0, idx1, out0, out1, si0, si1, ss0, ss1, sg):
    wid = lax.axis_index("s") * 2 + lax.axis_index("c")
    base = wid * W_ELEMS
    idx_bufs = (idx0, idx1)
    out_bufs = (out0, out1)
    si = (si0, si1)
    ss = (ss0, ss1)

    def wait_idx(b):
        pltpu.make_async_copy(lin_hbm.at[pl.ds(0, CHUNK)], idx_bufs[b],
                              si[b]).wait()

    def wait_store(b):
        pltpu.make_async_copy(out_bufs[b], out_hbm.at[pl.ds(0, CHUNK)],
                              ss[b]).wait()

    pltpu.async_copy(lin_hbm.at[pl.ds(base, CHUNK)], idx0, si0)

    def pair_body(i, carry):
        off0 = base + (2 * i) * CHUNK

        for b in range(2):
            off = off0 + b * CHUNK
            nxt = off + CHUNK
            if b == 0:
                pltpu.async_copy(lin_hbm.at[pl.ds(nxt, CHUNK)], idx1, si1)
            else:
                @pl.when(i < W_CHUNKS // 2 - 1)
                def _():
                    pltpu.async_copy(lin_hbm.at[pl.ds(nxt, CHUNK)], idx0,
                                     si0)
            wait_idx(b)

            @pl.when(i > 0)
            def _():
                wait_store(b)

            copies = [
                pltpu.async_copy(
                    occ_hbm.at[idx_bufs[b].at[pl.ds(r * 128, 128)]],
                    out_bufs[b].at[pl.ds(r * 128, 128)], sg)
                for r in range(GATHERS)
            ]
            for cp in copies:
                cp.wait()
            pltpu.async_copy(out_bufs[b], out_hbm.at[pl.ds(off, CHUNK)],
                             ss[b])
        return carry

    lax.fori_loop(0, W_CHUNKS // 2, pair_body, 0)
    wait_store(0)
    wait_store(1)


def _phase2(lin, occ_t_flat):
    mesh = plsc.VectorSubcoreMesh(core_axis_name="c", subcore_axis_name="s")
    k = functools.partial(
        pl.kernel,
        mesh=mesh,
        out_type=jax.ShapeDtypeStruct((N_PAD,), jnp.float32),
        scratch_types=[
            pltpu.VMEM((CHUNK,), jnp.int32),
            pltpu.VMEM((CHUNK,), jnp.int32),
            pltpu.VMEM((CHUNK,), jnp.float32),
            pltpu.VMEM((CHUNK,), jnp.float32),
            pltpu.SemaphoreType.DMA,
            pltpu.SemaphoreType.DMA,
            pltpu.SemaphoreType.DMA,
            pltpu.SemaphoreType.DMA,
            pltpu.SemaphoreType.DMA,
        ],
    )(_phase2_body)
    return k(lin, occ_t_flat)


def kernel(pts, bidx, ts, occ_grid, ts_keyframes):
    pts3 = pts.T.reshape(3, ROWS, 128)
    occ_t_flat = jnp.transpose(occ_grid, (1, 2, 3, 0)).reshape(-1)

    lin = _phase1(ts_keyframes, pts3, bidx, ts)
    out = _phase2(lin, occ_t_flat)
    return out[:N]


# speedup vs baseline: 1.0699x; 1.0699x over previous
import functools

import jax
import jax.numpy as jnp
from jax import lax
from jax.experimental import pallas as pl
from jax.experimental.pallas import tpu as pltpu
from jax.experimental.pallas import tpu_sc as plsc

NUM_BATCHES = 8
NUM_FRAMES = 16
RES = 64
N = 2_000_000
NBF = NUM_BATCHES * NUM_FRAMES
TOTAL_CELLS = NBF * RES * RES * RES

BLK = 32768
BLK_ROWS = BLK // 128
ROWS = N // 128
GRID1 = 62
N_PAD = GRID1 * BLK

SLICES = (16, 16, 16, 14)

NW = 32
CHUNK = 1024
GATHERS = CHUNK // 128


def _phase1_body(kf_ref, p_ref, b_ref, t_ref, o_ref):
    def cellq(v):
        g = ((v / 2.0 + 0.5) * float(RES)).astype(jnp.int32)
        return jnp.clip(g, 0, RES - 1)

    gx = cellq(p_ref[0])
    gy = cellq(p_ref[1])
    gz = cellq(p_ref[2])
    spatial = (gx * (RES * RES) + gy * RES + gz).reshape(BLK)

    t = t_ref[...]
    cell = jnp.clip((t * float(NUM_FRAMES - 1)).astype(jnp.int32), 0,
                    NUM_FRAMES - 2)
    left = jnp.zeros_like(t)
    right = jnp.zeros_like(t)
    for i in range(NUM_FRAMES):
        ki = kf_ref[i]
        if i <= NUM_FRAMES - 2:
            left = jnp.where(cell == i, ki, left)
        if i >= 1:
            right = jnp.where(cell == i - 1, ki, right)
    fidx = cell + jnp.where(jnp.abs(t - left) <= jnp.abs(right - t), 0, 1)

    lin = spatial * NBF + b_ref[...] * NUM_FRAMES + fidx
    o_ref[...] = jnp.clip(lin, 0, TOTAL_CELLS - 1)


def _phase1(kf, pts3, bs, t, off, nblk):
    blk1 = lambda: pl.BlockSpec((BLK,), lambda i: (i + off,))
    return pl.pallas_call(
        _phase1_body,
        grid=(nblk,),
        in_specs=[pl.BlockSpec(memory_space=pltpu.SMEM),
                  pl.BlockSpec((3, BLK_ROWS, 128), lambda i: (0, i + off, 0)),
                  blk1(), blk1()],
        out_specs=pl.BlockSpec((BLK,), lambda i: (i,)),
        out_shape=jax.ShapeDtypeStruct((nblk * BLK,), jnp.int32),
    )(kf, pts3, bs, t)


def _make_phase2_body(w_elems):
    w_chunks = w_elems // CHUNK

    def body(lin_hbm, occ_hbm, out_hbm,
             idx0, idx1, out0, out1, si0, si1, ss0, ss1, sg):
        wid = lax.axis_index("s") * 2 + lax.axis_index("c")
        base = wid * w_elems
        idx_bufs = (idx0, idx1)
        out_bufs = (out0, out1)
        si = (si0, si1)
        ss = (ss0, ss1)

        def wait_idx(b):
            pltpu.make_async_copy(lin_hbm.at[pl.ds(0, CHUNK)], idx_bufs[b],
                                  si[b]).wait()

        def wait_store(b):
            pltpu.make_async_copy(out_bufs[b], out_hbm.at[pl.ds(0, CHUNK)],
                                  ss[b]).wait()

        pltpu.async_copy(lin_hbm.at[pl.ds(base, CHUNK)], idx0, si0)

        def pair_body(i, carry):
            off0 = base + (2 * i) * CHUNK

            for b in range(2):
                off = off0 + b * CHUNK
                nxt = off + CHUNK
                if b == 0:
                    pltpu.async_copy(lin_hbm.at[pl.ds(nxt, CHUNK)], idx1,
                                     si1)
                else:
                    @pl.when(i < w_chunks // 2 - 1)
                    def _():
                        pltpu.async_copy(lin_hbm.at[pl.ds(nxt, CHUNK)],
                                         idx0, si0)
                wait_idx(b)

                @pl.when(i > 0)
                def _():
                    wait_store(b)

                copies = [
                    pltpu.async_copy(
                        occ_hbm.at[idx_bufs[b].at[pl.ds(r * 128, 128)]],
                        out_bufs[b].at[pl.ds(r * 128, 128)], sg)
                    for r in range(GATHERS)
                ]
                for cp in copies:
                    cp.wait()
                pltpu.async_copy(out_bufs[b],
                                 out_hbm.at[pl.ds(off, CHUNK)], ss[b])
            return carry

        lax.fori_loop(0, w_chunks // 2, pair_body, 0)
        wait_store(0)
        wait_store(1)

    return body


def _phase2(lin, occ_t_flat, n_elems):
    mesh = plsc.VectorSubcoreMesh(core_axis_name="c", subcore_axis_name="s")
    k = functools.partial(
        pl.kernel,
        mesh=mesh,
        out_type=jax.ShapeDtypeStruct((n_elems,), jnp.float32),
        scratch_types=[
            pltpu.VMEM((CHUNK,), jnp.int32),
            pltpu.VMEM((CHUNK,), jnp.int32),
            pltpu.VMEM((CHUNK,), jnp.float32),
            pltpu.VMEM((CHUNK,), jnp.float32),
            pltpu.SemaphoreType.DMA,
            pltpu.SemaphoreType.DMA,
            pltpu.SemaphoreType.DMA,
            pltpu.SemaphoreType.DMA,
            pltpu.SemaphoreType.DMA,
        ],
    )(_make_phase2_body(n_elems // NW))
    return k(lin, occ_t_flat)


def kernel(pts, bidx, ts, occ_grid, ts_keyframes):
    pts3 = pts.T.reshape(3, ROWS, 128)
    occ_t_flat = jnp.transpose(occ_grid, (1, 2, 3, 0)).reshape(-1)

    outs = []
    off = 0
    for nblk in SLICES:
        lin = _phase1(ts_keyframes, pts3, bidx, ts, off, nblk)
        outs.append(_phase2(lin, occ_t_flat, nblk * BLK))
        off += nblk
    return jnp.concatenate(outs)[:N]


# speedup vs baseline: 1.2690x; 1.1861x over previous
import functools

import jax
import jax.numpy as jnp
from jax import lax
from jax.experimental import pallas as pl
from jax.experimental.pallas import tpu as pltpu
from jax.experimental.pallas import tpu_sc as plsc

NUM_BATCHES = 8
NUM_FRAMES = 16
RES = 64
N = 2_000_000
NBF = NUM_BATCHES * NUM_FRAMES
TOTAL_CELLS = NBF * RES * RES * RES

BLK = 32768
BLK_ROWS = BLK // 128
ROWS = N // 128
GRID1 = 62
N_PAD = GRID1 * BLK

SLICES = (16, 16, 16, 14)

NW = 32
CHUNK = 1024
GATHERS = CHUNK // 128


def _phase1_body(kf_ref, p_ref, b_ref, t_ref, o_ref):
    def cellq(v):
        g = ((v / 2.0 + 0.5) * float(RES)).astype(jnp.int32)
        return jnp.clip(g, 0, RES - 1)

    gx = cellq(p_ref[0])
    gy = cellq(p_ref[1])
    gz = cellq(p_ref[2])
    spatial = (gx * (RES * RES) + gy * RES + gz).reshape(BLK)

    t = t_ref[...]
    cell = jnp.clip((t * float(NUM_FRAMES - 1)).astype(jnp.int32), 0,
                    NUM_FRAMES - 2)
    left = jnp.zeros_like(t)
    right = jnp.zeros_like(t)
    for i in range(NUM_FRAMES):
        ki = kf_ref[i]
        if i <= NUM_FRAMES - 2:
            left = jnp.where(cell == i, ki, left)
        if i >= 1:
            right = jnp.where(cell == i - 1, ki, right)
    fidx = cell + jnp.where(jnp.abs(t - left) <= jnp.abs(right - t), 0, 1)

    lin = spatial * NBF + b_ref[...] * NUM_FRAMES + fidx
    o_ref[...] = jnp.clip(lin, 0, TOTAL_CELLS - 1)


def _phase1(kf, pts3s, bs, t, off, nblk):
    blk1 = lambda: pl.BlockSpec((BLK,), lambda i: (i + off,))
    return pl.pallas_call(
        _phase1_body,
        grid=(nblk,),
        in_specs=[pl.BlockSpec(memory_space=pltpu.SMEM),
                  pl.BlockSpec((3, BLK_ROWS, 128), lambda i: (0, i, 0)),
                  blk1(), blk1()],
        out_specs=pl.BlockSpec((BLK,), lambda i: (i,)),
        out_shape=jax.ShapeDtypeStruct((nblk * BLK,), jnp.int32),
    )(kf, pts3s, bs, t)


def _make_phase2_body(w_elems):
    w_chunks = w_elems // CHUNK

    def body(lin_hbm, occ_hbm, out_hbm,
             idx0, idx1, out0, out1, si0, si1, ss0, ss1, sg):
        wid = lax.axis_index("s") * 2 + lax.axis_index("c")
        base = wid * w_elems
        idx_bufs = (idx0, idx1)
        out_bufs = (out0, out1)
        si = (si0, si1)
        ss = (ss0, ss1)

        def wait_idx(b):
            pltpu.make_async_copy(lin_hbm.at[pl.ds(0, CHUNK)], idx_bufs[b],
                                  si[b]).wait()

        def wait_store(b):
            pltpu.make_async_copy(out_bufs[b], out_hbm.at[pl.ds(0, CHUNK)],
                                  ss[b]).wait()

        pltpu.async_copy(lin_hbm.at[pl.ds(base, CHUNK)], idx0, si0)

        def pair_body(i, carry):
            off0 = base + (2 * i) * CHUNK

            for b in range(2):
                off = off0 + b * CHUNK
                nxt = off + CHUNK
                if b == 0:
                    pltpu.async_copy(lin_hbm.at[pl.ds(nxt, CHUNK)], idx1,
                                     si1)
                else:
                    @pl.when(i < w_chunks // 2 - 1)
                    def _():
                        pltpu.async_copy(lin_hbm.at[pl.ds(nxt, CHUNK)],
                                         idx0, si0)
                wait_idx(b)

                @pl.when(i > 0)
                def _():
                    wait_store(b)

                copies = [
                    pltpu.async_copy(
                        occ_hbm.at[idx_bufs[b].at[pl.ds(r * 128, 128)]],
                        out_bufs[b].at[pl.ds(r * 128, 128)], sg)
                    for r in range(GATHERS)
                ]
                for cp in copies:
                    cp.wait()
                pltpu.async_copy(out_bufs[b],
                                 out_hbm.at[pl.ds(off, CHUNK)], ss[b])
            return carry

        lax.fori_loop(0, w_chunks // 2, pair_body, 0)
        wait_store(0)
        wait_store(1)

    return body


def _phase2(lin, occ_t_flat, n_elems):
    mesh = plsc.VectorSubcoreMesh(core_axis_name="c", subcore_axis_name="s")
    k = functools.partial(
        pl.kernel,
        mesh=mesh,
        out_type=jax.ShapeDtypeStruct((n_elems,), jnp.float32),
        scratch_types=[
            pltpu.VMEM((CHUNK,), jnp.int32),
            pltpu.VMEM((CHUNK,), jnp.int32),
            pltpu.VMEM((CHUNK,), jnp.float32),
            pltpu.VMEM((CHUNK,), jnp.float32),
            pltpu.SemaphoreType.DMA,
            pltpu.SemaphoreType.DMA,
            pltpu.SemaphoreType.DMA,
            pltpu.SemaphoreType.DMA,
            pltpu.SemaphoreType.DMA,
        ],
    )(_make_phase2_body(n_elems // NW))
    return k(lin, occ_t_flat)


def kernel(pts, bidx, ts, occ_grid, ts_keyframes):
    ptsT = pts.T
    occ_t_flat = jnp.transpose(occ_grid, (1, 2, 3, 0)).reshape(-1)

    outs = []
    off = 0
    for nblk in SLICES:
        c0 = off * BLK
        c1 = min((off + nblk) * BLK, N)
        pts3s = ptsT[:, c0:c1].reshape(3, (c1 - c0) // 128, 128)
        lin = _phase1(ts_keyframes, pts3s, bidx, ts, off, nblk)
        outs.append(_phase2(lin, occ_t_flat, nblk * BLK))
        off += nblk
    return jnp.concatenate(outs)[:N]


# speedup vs baseline: 1.2952x; 1.0207x over previous
import functools

import jax
import jax.numpy as jnp
from jax import lax
from jax.experimental import pallas as pl
from jax.experimental.pallas import tpu as pltpu
from jax.experimental.pallas import tpu_sc as plsc

NUM_BATCHES = 8
NUM_FRAMES = 16
RES = 64
N = 2_000_000
NBF = NUM_BATCHES * NUM_FRAMES
TOTAL_CELLS = NBF * RES * RES * RES

BLK = 32768
BLK_ROWS = BLK // 128
ROWS = N // 128
GRID1 = 62
N_PAD = GRID1 * BLK

SLICES = (8, 18, 18, 18)

NW = 32
CHUNK = 1024
GATHERS = CHUNK // 128


def _phase1_body(kf_ref, p_ref, b_ref, t_ref, o_ref):
    def cellq(v):
        g = ((v / 2.0 + 0.5) * float(RES)).astype(jnp.int32)
        return jnp.clip(g, 0, RES - 1)

    gx = cellq(p_ref[0])
    gy = cellq(p_ref[1])
    gz = cellq(p_ref[2])
    spatial = (gx * (RES * RES) + gy * RES + gz).reshape(BLK)

    t = t_ref[...]
    cell = jnp.clip((t * float(NUM_FRAMES - 1)).astype(jnp.int32), 0,
                    NUM_FRAMES - 2)
    left = jnp.zeros_like(t)
    right = jnp.zeros_like(t)
    for i in range(NUM_FRAMES):
        ki = kf_ref[i]
        if i <= NUM_FRAMES - 2:
            left = jnp.where(cell == i, ki, left)
        if i >= 1:
            right = jnp.where(cell == i - 1, ki, right)
    fidx = cell + jnp.where(jnp.abs(t - left) <= jnp.abs(right - t), 0, 1)

    lin = spatial * NBF + b_ref[...] * NUM_FRAMES + fidx
    o_ref[...] = jnp.clip(lin, 0, TOTAL_CELLS - 1)


def _phase1(kf, pts3s, bs, t, off, nblk):
    blk1 = lambda: pl.BlockSpec((BLK,), lambda i: (i + off,))
    return pl.pallas_call(
        _phase1_body,
        grid=(nblk,),
        in_specs=[pl.BlockSpec(memory_space=pltpu.SMEM),
                  pl.BlockSpec((3, BLK_ROWS, 128), lambda i: (0, i, 0)),
                  blk1(), blk1()],
        out_specs=pl.BlockSpec((BLK,), lambda i: (i,)),
        out_shape=jax.ShapeDtypeStruct((nblk * BLK,), jnp.int32),
    )(kf, pts3s, bs, t)


def _make_phase2_body(w_elems):
    w_chunks = w_elems // CHUNK

    def body(lin_hbm, occ_hbm, out_hbm,
             idx0, idx1, out0, out1, si0, si1, ss0, ss1, sg):
        wid = lax.axis_index("s") * 2 + lax.axis_index("c")
        base = wid * w_elems
        idx_bufs = (idx0, idx1)
        out_bufs = (out0, out1)
        si = (si0, si1)
        ss = (ss0, ss1)

        def wait_idx(b):
            pltpu.make_async_copy(lin_hbm.at[pl.ds(0, CHUNK)], idx_bufs[b],
                                  si[b]).wait()

        def wait_store(b):
            pltpu.make_async_copy(out_bufs[b], out_hbm.at[pl.ds(0, CHUNK)],
                                  ss[b]).wait()

        pltpu.async_copy(lin_hbm.at[pl.ds(base, CHUNK)], idx0, si0)

        def pair_body(i, carry):
            off0 = base + (2 * i) * CHUNK

            for b in range(2):
                off = off0 + b * CHUNK
                nxt = off + CHUNK
                if b == 0:
                    pltpu.async_copy(lin_hbm.at[pl.ds(nxt, CHUNK)], idx1,
                                     si1)
                else:
                    @pl.when(i < w_chunks // 2 - 1)
                    def _():
                        pltpu.async_copy(lin_hbm.at[pl.ds(nxt, CHUNK)],
                                         idx0, si0)
                wait_idx(b)

                @pl.when(i > 0)
                def _():
                    wait_store(b)

                copies = [
                    pltpu.async_copy(
                        occ_hbm.at[idx_bufs[b].at[pl.ds(r * 128, 128)]],
                        out_bufs[b].at[pl.ds(r * 128, 128)], sg)
                    for r in range(GATHERS)
                ]
                for cp in copies:
                    cp.wait()
                pltpu.async_copy(out_bufs[b],
                                 out_hbm.at[pl.ds(off, CHUNK)], ss[b])
            return carry

        lax.fori_loop(0, w_chunks // 2, pair_body, 0)
        wait_store(0)
        wait_store(1)

    return body


def _phase2(lin, occ_t_flat, n_elems):
    mesh = plsc.VectorSubcoreMesh(core_axis_name="c", subcore_axis_name="s")
    k = functools.partial(
        pl.kernel,
        mesh=mesh,
        out_type=jax.ShapeDtypeStruct((n_elems,), jnp.float32),
        scratch_types=[
            pltpu.VMEM((CHUNK,), jnp.int32),
            pltpu.VMEM((CHUNK,), jnp.int32),
            pltpu.VMEM((CHUNK,), jnp.float32),
            pltpu.VMEM((CHUNK,), jnp.float32),
            pltpu.SemaphoreType.DMA,
            pltpu.SemaphoreType.DMA,
            pltpu.SemaphoreType.DMA,
            pltpu.SemaphoreType.DMA,
            pltpu.SemaphoreType.DMA,
        ],
    )(_make_phase2_body(n_elems // NW))
    return k(lin, occ_t_flat)


def kernel(pts, bidx, ts, occ_grid, ts_keyframes):
    ptsT = pts.T
    occ_t_flat = jnp.transpose(occ_grid, (1, 2, 3, 0)).reshape(-1)

    outs = []
    off = 0
    for nblk in SLICES:
        c0 = off * BLK
        c1 = min((off + nblk) * BLK, N)
        pts3s = ptsT[:, c0:c1].reshape(3, (c1 - c0) // 128, 128)
        lin = _phase1(ts_keyframes, pts3s, bidx, ts, off, nblk)
        outs.append(_phase2(lin, occ_t_flat, nblk * BLK))
        off += nblk
    return jnp.concatenate(outs)[:N]
